# baseline (speedup 1.0000x reference)
import jax
import jax.numpy as jnp
from jax import lax
from jax.experimental import pallas as pl
from jax.experimental.pallas import tpu as pltpu

NUM_ROWS = 1000000
BATCH = 16384
EMBED = 64
HIDDEN = 128

_GBS = 2048
_WIN = 64


def _body(uid_ref, bid_ref, utbl, btbl, w1t_ref, b1_ref, w2t_ref, b2_ref,
          out_ref, gu_v, gb_v, usem, bsem):
    i = pl.program_id(0)
    base = i * _GBS

    def start_one(k):
        u = uid_ref[base + k]
        b = bid_ref[base + k]
        pltpu.make_async_copy(
            utbl.at[pl.ds(u, 1)], gu_v.at[pl.ds(k, 1)], usem).start()
        pltpu.make_async_copy(
            btbl.at[pl.ds(b, 1)], gb_v.at[pl.ds(k, 1)], bsem).start()

    def wait_one():
        pltpu.make_async_copy(
            utbl.at[pl.ds(0, 1)], gu_v.at[pl.ds(0, 1)], usem).wait()
        pltpu.make_async_copy(
            btbl.at[pl.ds(0, 1)], gb_v.at[pl.ds(0, 1)], bsem).wait()

    lax.fori_loop(0, _WIN, lambda k, c: (start_one(k), c)[1], 0, unroll=8)

    def steady(k, c):
        wait_one()
        start_one(k + _WIN)
        return c

    lax.fori_loop(0, _GBS - _WIN, steady, 0, unroll=8)
    lax.fori_loop(0, _WIN, lambda k, c: (wait_one(), c)[1], 0, unroll=8)

    w1t = w1t_ref[:]
    h = (jnp.dot(gu_v[:], w1t[:EMBED], preferred_element_type=jnp.float32)
         + jnp.dot(gb_v[:], w1t[EMBED:], preferred_element_type=jnp.float32)
         + b1_ref[:])
    h = jnp.where(h >= 0, h, 0.01 * h)
    raw = jnp.dot(h, w2t_ref[:], preferred_element_type=jnp.float32) + b2_ref[0, 0]
    out_ref[:] = 1.0 + 4.0 * jax.nn.sigmoid(raw)


_fused = pl.pallas_call(
    _body,
    grid_spec=pltpu.PrefetchScalarGridSpec(
        num_scalar_prefetch=2,
        grid=(BATCH // _GBS,),
        in_specs=[
            pl.BlockSpec(memory_space=pl.ANY),
            pl.BlockSpec(memory_space=pl.ANY),
            pl.BlockSpec((HIDDEN, HIDDEN), lambda i, uid, bid: (0, 0)),
            pl.BlockSpec((1, HIDDEN), lambda i, uid, bid: (0, 0)),
            pl.BlockSpec((HIDDEN, 1), lambda i, uid, bid: (0, 0)),
            pl.BlockSpec(memory_space=pltpu.SMEM),
        ],
        out_specs=pl.BlockSpec((_GBS, 1), lambda i, uid, bid: (i, 0)),
        scratch_shapes=[
            pltpu.VMEM((_GBS, EMBED), jnp.float32),
            pltpu.VMEM((_GBS, EMBED), jnp.float32),
            pltpu.SemaphoreType.DMA,
            pltpu.SemaphoreType.DMA,
        ],
    ),
    out_shape=jax.ShapeDtypeStruct((BATCH, 1), jnp.float32),
)


def kernel(user_id, book_id, user_emb, book_emb, W1, b1, W2, b2):
    uid = user_id.astype(jnp.int32)
    bid = book_id.astype(jnp.int32)
    return _fused(uid, bid, user_emb, book_emb,
                  W1.swapaxes(0, 1), b1.reshape(1, HIDDEN),
                  W2.swapaxes(0, 1), b2.reshape(1, 1))
